# baseline (speedup 1.0000x reference)
import functools

import jax
import jax.numpy as jnp
from jax.experimental import pallas as pl
from jax.experimental.pallas import tpu as pltpu

N_E = 8192
E_DIM = 32
BETA = 0.25

TN = 2048
TK = 1024
KB = N_E // TK


def _vq_body(z_ref, w_ref, idx_ref, zq_ref, loss_ref, perp_ref,
             cnt_ref, lacc_ref, *, n_tokens, nb):
    i = pl.program_id(0)

    @pl.when(i == 0)
    def _init():
        cnt_ref[...] = jnp.zeros_like(cnt_ref)
        lacc_ref[0, 0] = 0.0

    zb = z_ref[...]
    idx = idx_ref[...]

    zq = jnp.zeros((TN, E_DIM), dtype=jnp.float32)
    for kc in range(KB):
        iota = jax.lax.broadcasted_iota(jnp.int32, (TN, TK), 1) + kc * TK
        oh = (iota == idx).astype(jnp.float32)
        wc = w_ref[pl.ds(kc * TK, TK), :]
        zq = zq + jax.lax.dot_general(
            oh, wc, (((1,), (0,)), ((), ())),
            preferred_element_type=jnp.float32)
        cnt_ref[:, pl.ds(kc * TK, TK)] += jnp.sum(oh, axis=0, keepdims=True)

    zq_ref[...] = zb + (zq - zb)

    diff = zq - zb
    lacc_ref[0, 0] += jnp.sum(diff * diff)

    @pl.when(i == nb - 1)
    def _finalize():
        m1 = lacc_ref[0, 0] / jnp.float32(n_tokens * E_DIM)
        loss_ref[...] = jnp.full((1, 1), BETA * m1 + m1, dtype=jnp.float32)
        e_mean = cnt_ref[...] / jnp.float32(n_tokens)
        perp_ref[...] = jnp.exp(
            -jnp.sum(e_mean * jnp.log(e_mean + 1e-10), keepdims=True))


@jax.jit
def kernel(z, weight):
    zp = jnp.transpose(z, (0, 2, 3, 1))
    z_flat = zp.reshape(-1, E_DIM)
    n = z_flat.shape[0]
    nb = n // TN

    d = (jnp.sum(z_flat ** 2, axis=1, keepdims=True)
         + jnp.sum(weight ** 2, axis=1)
         - 2.0 * jnp.matmul(z_flat, weight.T))
    mei = jnp.argmin(d, axis=1)
    enc_t = (jax.lax.broadcasted_iota(jnp.int32, (N_E, n), 0) == mei[None, :]
             ).astype(jnp.float32)
    enc = enc_t.T

    body = functools.partial(_vq_body, n_tokens=n, nb=nb)
    zq_st, loss, perp = pl.pallas_call(
        body,
        grid=(nb,),
        in_specs=[
            pl.BlockSpec((TN, E_DIM), lambda i: (i, 0)),
            pl.BlockSpec((N_E, E_DIM), lambda i: (0, 0)),
            pl.BlockSpec((TN, 1), lambda i: (i, 0)),
        ],
        out_specs=[
            pl.BlockSpec((TN, E_DIM), lambda i: (i, 0)),
            pl.BlockSpec((1, 1), lambda i: (0, 0)),
            pl.BlockSpec((1, 1), lambda i: (0, 0)),
        ],
        out_shape=[
            jax.ShapeDtypeStruct((n, E_DIM), jnp.float32),
            jax.ShapeDtypeStruct((1, 1), jnp.float32),
            jax.ShapeDtypeStruct((1, 1), jnp.float32),
        ],
        scratch_shapes=[
            pltpu.VMEM((1, N_E), jnp.float32),
            pltpu.SMEM((1, 1), jnp.float32),
        ],
    )(z_flat, weight, mei[:, None].astype(jnp.int32))

    z_q_out = jnp.transpose(zq_st.reshape(zp.shape), (0, 3, 1, 2))
    return (loss[0, 0], z_q_out, perp[0, 0], enc, mei[:, None])


# speedup vs baseline: 1.0025x; 1.0025x over previous
import functools

import jax
import jax.numpy as jnp
from jax.experimental import pallas as pl
from jax.experimental.pallas import tpu as pltpu

N_E = 8192
E_DIM = 32
BETA = 0.25

TN = 1024
TK = 1024
KB = N_E // TK


def _vq_body(z_ref, w_ref, idx_ref, zq_ref, loss_ref, perp_ref,
             cnt_ref, lacc_ref, *, n_tokens, nb):
    i = pl.program_id(0)

    @pl.when(i == 0)
    def _init():
        cnt_ref[...] = jnp.zeros_like(cnt_ref)
        lacc_ref[0, 0] = 0.0

    zb = z_ref[...]
    idx = idx_ref[...]

    zq = jnp.zeros((TN, E_DIM), dtype=jnp.float32)
    for kc in range(KB):
        iota = jax.lax.broadcasted_iota(jnp.int32, (TN, TK), 1) + kc * TK
        oh = (iota == idx).astype(jnp.float32)
        wc = w_ref[pl.ds(kc * TK, TK), :]
        zq = zq + jax.lax.dot_general(
            oh, wc, (((1,), (0,)), ((), ())),
            preferred_element_type=jnp.float32)
        cnt_ref[:, pl.ds(kc * TK, TK)] += jnp.sum(oh, axis=0, keepdims=True)

    zq_ref[...] = zb + (zq - zb)

    diff = zq - zb
    lacc_ref[0, 0] += jnp.sum(diff * diff)

    @pl.when(i == nb - 1)
    def _finalize():
        m1 = lacc_ref[0, 0] / jnp.float32(n_tokens * E_DIM)
        loss_ref[...] = jnp.full((1, 1), BETA * m1 + m1, dtype=jnp.float32)
        e_mean = cnt_ref[...] / jnp.float32(n_tokens)
        perp_ref[...] = jnp.exp(
            -jnp.sum(e_mean * jnp.log(e_mean + 1e-10), keepdims=True))


@jax.jit
def kernel(z, weight):
    zp = jnp.transpose(z, (0, 2, 3, 1))
    z_flat = zp.reshape(-1, E_DIM)
    n = z_flat.shape[0]
    nb = n // TN

    d = (jnp.sum(z_flat ** 2, axis=1, keepdims=True)
         + jnp.sum(weight ** 2, axis=1)
         - 2.0 * jnp.matmul(z_flat, weight.T))
    mei = jnp.argmin(d, axis=1)
    enc_t = (jax.lax.broadcasted_iota(jnp.int32, (N_E, n), 0) == mei[None, :]
             ).astype(jnp.float32)
    enc = enc_t.T

    body = functools.partial(_vq_body, n_tokens=n, nb=nb)
    zq_st, loss, perp = pl.pallas_call(
        body,
        grid=(nb,),
        in_specs=[
            pl.BlockSpec((TN, E_DIM), lambda i: (i, 0)),
            pl.BlockSpec((N_E, E_DIM), lambda i: (0, 0)),
            pl.BlockSpec((TN, 1), lambda i: (i, 0)),
        ],
        out_specs=[
            pl.BlockSpec((TN, E_DIM), lambda i: (i, 0)),
            pl.BlockSpec((1, 1), lambda i: (0, 0)),
            pl.BlockSpec((1, 1), lambda i: (0, 0)),
        ],
        out_shape=[
            jax.ShapeDtypeStruct((n, E_DIM), jnp.float32),
            jax.ShapeDtypeStruct((1, 1), jnp.float32),
            jax.ShapeDtypeStruct((1, 1), jnp.float32),
        ],
        scratch_shapes=[
            pltpu.VMEM((1, N_E), jnp.float32),
            pltpu.SMEM((1, 1), jnp.float32),
        ],
    )(z_flat, weight, mei[:, None].astype(jnp.int32))

    z_q_out = jnp.transpose(zq_st.reshape(zp.shape), (0, 3, 1, 2))
    return (loss[0, 0], z_q_out, perp[0, 0], enc, mei[:, None])
